# baseline (speedup 1.0000x reference)
import math

import jax
import jax.numpy as jnp
from jax import lax
from jax.experimental import pallas as pl
from jax.experimental.pallas import tpu as pltpu
from jax.experimental.pallas import tpu_sc as plsc

_SIZE = 100000
_N = 1024
_SMOOTHING = 0.1
_CONF = 1.0 - _SMOOTHING
_FILL = _SMOOTHING / (_SIZE - 2)
_C_ROW = (_SIZE - 2) * _FILL * math.log(_FILL) + _CONF * math.log(_CONF)

_NC = 2
_NS = 16
_LANES = 16
_NW = _NC * _NS

_R = 512
_BS = (_N - _R) // _NW
_NSTRIPE = _BS // 8

_CC = 6144
_NCOL = 16
_CA = _CC * _NCOL
_CW = _SIZE - _CA
_UNROLL = 24
_NACC = 8

_BR = 32
_NBUF = 4
_NSTEPS = _R // _BR


def _sc_body(xb, tgt, out, tgt_v, buf, part_v, sems):
    wid = lax.axis_index("s") * _NC + lax.axis_index("c")
    rbase = wid * _BS
    pltpu.sync_copy(tgt.at[pl.ds(_R + rbase, _BS)], tgt_v)
    t_all = tgt_v[pl.ds(0, _LANES)]

    def _chunk_copy(s, c):
        return pltpu.make_async_copy(
            xb.at[pl.ds(rbase + s * 8, 8), pl.ds(c * _CC, _CC)],
            buf.at[lax.rem(c, 2)],
            sems.at[lax.rem(c, 2)],
        )

    iota = lax.iota(jnp.int32, _LANES)
    onehot0 = jnp.where(iota == 0, jnp.float32(1.0), jnp.float32(0.0))
    zero = jnp.zeros((_LANES,), jnp.float32)
    one = jnp.float32(1.0)
    zf = jnp.float32(0.0)

    total_v = zero
    _chunk_copy(0, jnp.int32(0)).start()
    for s in range(_NSTRIPE):
        def _col_body(c, total_v, s=s):
            @pl.when(c + 1 < _NCOL)
            def _pre():
                _chunk_copy(s, c + 1).start()

            if s + 1 < _NSTRIPE:
                @pl.when(c + 1 == _NCOL)
                def _pre2():
                    _chunk_copy(s + 1, jnp.int32(0)).start()

            _chunk_copy(s, c).wait()
            p = lax.rem(c, 2)
            cfac = jnp.where(c == 0, one, zf)
            for r in range(8):
                t = t_all[s * 8 + r]
                mf = jnp.where(t != 0, one, zf)
                x0 = buf[p, r, pl.ds(0, _LANES)][0]

                def _body(i, carry, r=r, c=c):
                    accs, sel = carry
                    new = list(accs)
                    o = i * (_UNROLL * _LANES)
                    for u in range(_UNROLL):
                        v = buf[p, r, pl.ds(o + u * _LANES, _LANES)]
                        new[u % _NACC] = new[u % _NACC] + v
                        pos = (c * _CC + u * _LANES) + o + iota
                        sel = sel + jnp.where(pos == t, v, zf)
                    return tuple(new), sel

                accs, sel = lax.fori_loop(
                    0, _CC // (_UNROLL * _LANES), _body, ((zero,) * _NACC, zero)
                )
                rowacc = ((accs[0] + accs[1]) + (accs[2] + accs[3])) + (
                    (accs[4] + accs[5]) + (accs[6] + accs[7])
                )
                contrib = (
                    jnp.float32(-_FILL) * rowacc
                    - jnp.float32(_CONF - _FILL) * sel
                    + cfac
                    * (jnp.float32(_C_ROW) + jnp.float32(_FILL) * x0)
                    * onehot0
                )
                total_v = total_v + mf * contrib
            return total_v

        total_v = lax.fori_loop(0, _NCOL, _col_body, total_v)

    part_v[...] = total_v
    pltpu.sync_copy(part_v, out.at[wid])


def _sc_loss_partials(xb, target):
    return pl.kernel(
        _sc_body,
        out_type=jax.ShapeDtypeStruct((_NW, _LANES), jnp.float32),
        mesh=plsc.VectorSubcoreMesh(core_axis_name="c", subcore_axis_name="s"),
        scratch_types=[
            pltpu.VMEM((_BS,), jnp.int32),
            pltpu.VMEM((2, 8, _CC), jnp.float32),
            pltpu.VMEM((_LANES,), jnp.float32),
            pltpu.SemaphoreType.DMA((2,)),
        ],
    )(xb, target)


def _start_copy(x_ref, buf_ref, sems, step):
    slot = lax.rem(step, _NBUF)
    pltpu.make_async_copy(
        x_ref.at[pl.ds(step * _BR, _BR), :],
        buf_ref.at[slot],
        sems.at[slot],
    ).start()


def _tc_body(tgt_ref, x_ref, out_ref, buf_ref, sems, acc_ref):
    j = pl.program_id(0)

    @pl.when(j == 0)
    def _init():
        acc_ref[0] = jnp.float32(0.0)
        for k in range(_NBUF - 1):
            _start_copy(x_ref, buf_ref, sems, k)

    @pl.when(j + _NBUF - 1 < _NSTEPS)
    def _prefetch():
        _start_copy(x_ref, buf_ref, sems, j + _NBUF - 1)

    slot = lax.rem(j, _NBUF)
    pltpu.make_async_copy(
        x_ref.at[pl.ds(j * _BR, _BR), :], buf_ref.at[slot], sems.at[slot]
    ).wait()
    t_blk = tgt_ref[pl.ds(j * _BR, _BR), :]
    mask = t_blk != 0
    xb = buf_ref[slot]
    jj = lax.broadcasted_iota(jnp.int32, (_BR, _SIZE), 1)
    xt = jnp.sum(jnp.where(jj == t_blk, xb, jnp.float32(0.0)), axis=1,
                 keepdims=True)
    x0 = xb[:, 0:1]
    rterm = (
        jnp.float32(_C_ROW)
        + jnp.float32(_FILL) * x0
        - jnp.float32(_CONF - _FILL) * xt
    )
    acc_ref[0] += jnp.sum(
        jnp.where(mask, rterm, jnp.float32(0.0))
    ) - jnp.float32(_FILL) * jnp.sum(jnp.where(mask, xb, jnp.float32(0.0)))

    @pl.when(j == _NSTEPS - 1)
    def _finish():
        out_ref[...] = jnp.broadcast_to(acc_ref[0], (1, 1))


def _tc_top_loss(x_top, tgt2d_top):
    return pl.pallas_call(
        _tc_body,
        grid=(_NSTEPS,),
        in_specs=[
            pl.BlockSpec((_R, 1), lambda j: (0, 0)),
            pl.BlockSpec(memory_space=pl.ANY),
        ],
        out_specs=pl.BlockSpec((1, 1), lambda j: (0, 0)),
        out_shape=jax.ShapeDtypeStruct((1, 1), jnp.float32),
        scratch_shapes=[
            pltpu.VMEM((_NBUF, _BR, _SIZE), jnp.float32),
            pltpu.SemaphoreType.DMA((_NBUF,)),
            pltpu.SMEM((1,), jnp.float32),
        ],
        compiler_params=pltpu.CompilerParams(
            dimension_semantics=("arbitrary",),
        ),
    )(tgt2d_top, x_top)


def _corner_body(tgt_ref, xc_ref, out_ref):
    t_blk = tgt_ref[...]
    mask = t_blk != 0
    xb = xc_ref[...]
    jj = _CA + lax.broadcasted_iota(jnp.int32, (_N - _R, _CW), 1)
    xt = jnp.sum(jnp.where(jj == t_blk, xb, jnp.float32(0.0)), axis=1,
                 keepdims=True)
    part = -jnp.float32(_CONF - _FILL) * xt - jnp.float32(_FILL) * xb.sum(
        axis=1, keepdims=True
    )
    out_ref[...] = jnp.broadcast_to(
        jnp.sum(jnp.where(mask, part, jnp.float32(0.0))), (1, 1)
    )


def _tc_corner_loss(x_corner, tgt2d_bot):
    return pl.pallas_call(
        _corner_body,
        out_shape=jax.ShapeDtypeStruct((1, 1), jnp.float32),
    )(tgt2d_bot, x_corner)


def _finish_body(p_ref, s_ref, c_ref, out_ref):
    loss = jnp.sum(p_ref[...]) + jnp.sum(s_ref[...]) + jnp.sum(c_ref[...])
    out_ref[...] = jnp.broadcast_to(loss, (1, 1))


def _tc_finish(partials, s_top, s_corner):
    return pl.pallas_call(
        _finish_body,
        out_shape=jax.ShapeDtypeStruct((1, 1), jnp.float32),
    )(partials, s_top, s_corner)


def kernel(x, target):
    partials = _sc_loss_partials(x[_R:], target)
    s_top = _tc_top_loss(x[:_R], target[:_R].reshape(_R, 1))
    s_corner = _tc_corner_loss(x[_R:, _CA:], target[_R:].reshape(_N - _R, 1))
    out = _tc_finish(partials, s_top, s_corner)
    return out[0, 0]


# speedup vs baseline: 1.4944x; 1.4944x over previous
import math

import jax
import jax.numpy as jnp
from jax import lax
from jax.experimental import pallas as pl
from jax.experimental.pallas import tpu as pltpu
from jax.experimental.pallas import tpu_sc as plsc

_SIZE = 100000
_N = 1024
_SMOOTHING = 0.1
_CONF = 1.0 - _SMOOTHING
_FILL = _SMOOTHING / (_SIZE - 2)
_C_ROW = (_SIZE - 2) * _FILL * math.log(_FILL) + _CONF * math.log(_CONF)

_NC = 2
_NS = 16
_LANES = 16
_NW = _NC * _NS

_R = 512
_BS = (_N - _R) // _NW
_NSTRIPE = _BS // 8

_CC = 6144
_NCOL = 16
_CA = _CC * _NCOL
_CW = _SIZE - _CA
_UNROLL = 24
_NACC = 8

_BR = 32
_NBUF = 4
_NSTEPS = _R // _BR


def _sc_body(x, tgt, out, tgt_v, buf, part_v, sems):
    wid = lax.axis_index("s") * _NC + lax.axis_index("c")
    rbase = _R + wid * _BS
    pltpu.sync_copy(tgt.at[pl.ds(rbase, _BS)], tgt_v)
    t_all = tgt_v[pl.ds(0, _LANES)]

    def _chunk_copy(s, c):
        return pltpu.make_async_copy(
            x.at[pl.ds(rbase + s * 8, 8), pl.ds(c * _CC, _CC)],
            buf.at[lax.rem(c, 2)],
            sems.at[lax.rem(c, 2)],
        )

    iota = lax.iota(jnp.int32, _LANES)
    onehot0 = jnp.where(iota == 0, jnp.float32(1.0), jnp.float32(0.0))
    zero = jnp.zeros((_LANES,), jnp.float32)
    one = jnp.float32(1.0)
    zf = jnp.float32(0.0)

    total_v = zero
    _chunk_copy(0, jnp.int32(0)).start()
    for s in range(_NSTRIPE):
        def _col_body(c, total_v, s=s):
            @pl.when(c + 1 < _NCOL)
            def _pre():
                _chunk_copy(s, c + 1).start()

            if s + 1 < _NSTRIPE:
                @pl.when(c + 1 == _NCOL)
                def _pre2():
                    _chunk_copy(s + 1, jnp.int32(0)).start()

            _chunk_copy(s, c).wait()
            p = lax.rem(c, 2)
            cfac = jnp.where(c == 0, one, zf)
            for r in range(8):
                t = t_all[s * 8 + r]
                mf = jnp.where(t != 0, one, zf)
                x0 = buf[p, r, pl.ds(0, _LANES)][0]

                def _body(i, carry, r=r, c=c):
                    accs, sel = carry
                    new = list(accs)
                    o = i * (_UNROLL * _LANES)
                    for u in range(_UNROLL):
                        v = buf[p, r, pl.ds(o + u * _LANES, _LANES)]
                        new[u % _NACC] = new[u % _NACC] + v
                        pos = (c * _CC + u * _LANES) + o + iota
                        sel = sel + jnp.where(pos == t, v, zf)
                    return tuple(new), sel

                accs, sel = lax.fori_loop(
                    0, _CC // (_UNROLL * _LANES), _body, ((zero,) * _NACC, zero)
                )
                rowacc = ((accs[0] + accs[1]) + (accs[2] + accs[3])) + (
                    (accs[4] + accs[5]) + (accs[6] + accs[7])
                )
                contrib = (
                    jnp.float32(-_FILL) * rowacc
                    - jnp.float32(_CONF - _FILL) * sel
                    + cfac
                    * (jnp.float32(_C_ROW) + jnp.float32(_FILL) * x0)
                    * onehot0
                )
                total_v = total_v + mf * contrib
            return total_v

        total_v = lax.fori_loop(0, _NCOL, _col_body, total_v)

    part_v[...] = total_v
    pltpu.sync_copy(part_v, out.at[wid])


def _sc_loss_partials(x, target):
    return pl.kernel(
        _sc_body,
        out_type=jax.ShapeDtypeStruct((_NW, _LANES), jnp.float32),
        mesh=plsc.VectorSubcoreMesh(core_axis_name="c", subcore_axis_name="s"),
        scratch_types=[
            pltpu.VMEM((_BS,), jnp.int32),
            pltpu.VMEM((2, 8, _CC), jnp.float32),
            pltpu.VMEM((_LANES,), jnp.float32),
            pltpu.SemaphoreType.DMA((2,)),
        ],
    )(x, target)


def _start_copy(x_ref, buf_ref, sems, step):
    slot = lax.rem(step, _NBUF)
    pltpu.make_async_copy(
        x_ref.at[pl.ds(step * _BR, _BR), :],
        buf_ref.at[slot],
        sems.at[slot],
    ).start()


def _tc_body(tgt_ref, x_ref, out_ref, buf_ref, sems, acc_ref):
    j = pl.program_id(0)

    @pl.when(j == 0)
    def _init():
        acc_ref[0] = jnp.float32(0.0)
        for k in range(_NBUF - 1):
            _start_copy(x_ref, buf_ref, sems, k)

    @pl.when(j + _NBUF - 1 < _NSTEPS)
    def _prefetch():
        _start_copy(x_ref, buf_ref, sems, j + _NBUF - 1)

    slot = lax.rem(j, _NBUF)
    pltpu.make_async_copy(
        x_ref.at[pl.ds(j * _BR, _BR), :], buf_ref.at[slot], sems.at[slot]
    ).wait()
    t_blk = tgt_ref[pl.ds(j * _BR, _BR), :]
    mask = t_blk != 0
    xb = buf_ref[slot]
    jj = lax.broadcasted_iota(jnp.int32, (_BR, _SIZE), 1)
    xt = jnp.sum(jnp.where(jj == t_blk, xb, jnp.float32(0.0)), axis=1,
                 keepdims=True)
    x0 = xb[:, 0:1]
    rterm = (
        jnp.float32(_C_ROW)
        + jnp.float32(_FILL) * x0
        - jnp.float32(_CONF - _FILL) * xt
    )
    acc_ref[0] += jnp.sum(
        jnp.where(mask, rterm, jnp.float32(0.0))
    ) - jnp.float32(_FILL) * jnp.sum(jnp.where(mask, xb, jnp.float32(0.0)))

    @pl.when(j == _NSTEPS - 1)
    def _finish():
        out_ref[...] = jnp.broadcast_to(acc_ref[0], (1, 1))


def _tc_top_loss(x_full, tgt2d):
    return pl.pallas_call(
        _tc_body,
        grid=(_NSTEPS,),
        in_specs=[
            pl.BlockSpec((_N, 1), lambda j: (0, 0)),
            pl.BlockSpec(memory_space=pl.ANY),
        ],
        out_specs=pl.BlockSpec((1, 1), lambda j: (0, 0)),
        out_shape=jax.ShapeDtypeStruct((1, 1), jnp.float32),
        scratch_shapes=[
            pltpu.VMEM((_NBUF, _BR, _SIZE), jnp.float32),
            pltpu.SemaphoreType.DMA((_NBUF,)),
            pltpu.SMEM((1,), jnp.float32),
        ],
        compiler_params=pltpu.CompilerParams(
            dimension_semantics=("arbitrary",),
        ),
    )(tgt2d, x_full)


def _corner_body(tgt_ref, xc_ref, out_ref):
    t_blk = tgt_ref[...]
    mask = t_blk != 0
    xb = xc_ref[...]
    jj = _CA + lax.broadcasted_iota(jnp.int32, (_N - _R, _CW), 1)
    xt = jnp.sum(jnp.where(jj == t_blk, xb, jnp.float32(0.0)), axis=1,
                 keepdims=True)
    part = -jnp.float32(_CONF - _FILL) * xt - jnp.float32(_FILL) * xb.sum(
        axis=1, keepdims=True
    )
    out_ref[...] = jnp.broadcast_to(
        jnp.sum(jnp.where(mask, part, jnp.float32(0.0))), (1, 1)
    )


def _tc_corner_loss(x_corner, tgt2d_bot):
    return pl.pallas_call(
        _corner_body,
        out_shape=jax.ShapeDtypeStruct((1, 1), jnp.float32),
    )(tgt2d_bot, x_corner)


def _finish_body(p_ref, s_ref, c_ref, out_ref):
    loss = jnp.sum(p_ref[...]) + jnp.sum(s_ref[...]) + jnp.sum(c_ref[...])
    out_ref[...] = jnp.broadcast_to(loss, (1, 1))


def _tc_finish(partials, s_top, s_corner):
    return pl.pallas_call(
        _finish_body,
        out_shape=jax.ShapeDtypeStruct((1, 1), jnp.float32),
    )(partials, s_top, s_corner)


def kernel(x, target):
    partials = _sc_loss_partials(x, target)
    s_top = _tc_top_loss(x, target.reshape(_N, 1))
    s_corner = _tc_corner_loss(x[_R:, _CA:], target[_R:].reshape(_N - _R, 1))
    out = _tc_finish(partials, s_top, s_corner)
    return out[0, 0]
